# baseline (speedup 1.0000x reference)
import functools

import jax
import jax.numpy as jnp
from jax import lax
from jax.experimental import pallas as pl
from jax.experimental.pallas import tpu as pltpu
from jax.experimental.pallas import tpu_sc as plsc

EMB = 64
N_NODE = 50000
N_EDGE = 800000
ROWW = 125
EROWS = N_EDGE // ROWW
N_TILE = 16
ROWS_PER_TILE = EROWS // N_TILE
CHUNK_ROWS = 8
N_CHUNK = ROWS_PER_TILE // CHUNK_ROWS
CE = CHUNK_ROWS * ROWW
NBLK = 10
BLK = N_NODE // NBLK
EPS = 1e-5
QTR = N_NODE // 4
QPAD = QTR + 12

_mesh = plsc.VectorSubcoreMesh(core_axis_name="c", subcore_axis_name="s")
f32 = jnp.float32
i32 = jnp.int32


def _instats_part_body(cx_ref, vx_ref, ef_ref, out_ref):
    cx = cx_ref[...]
    vx = vx_ref[...]
    ef = ef_ref[...]
    z = lambda n: jnp.zeros((1, n), f32)
    w = jnp.where(pl.program_id(0) == 0, 1.0, 0.0).astype(f32)
    out_ref[...] = jnp.concatenate([
        jnp.sum(cx, axis=0, keepdims=True), z(3),
        jnp.sum(cx * cx, axis=0, keepdims=True), z(3),
        jnp.sum(vx, axis=0, keepdims=True), z(5),
        jnp.sum(vx * vx, axis=0, keepdims=True), z(5),
        (w * jnp.sum(ef)).reshape(1, 1), z(7),
        (w * jnp.sum(ef * ef)).reshape(1, 1), z(55)], axis=1)[None]


def _instats_fin_body(p_ref, eg_ref, eb_ref, cs_ref, vs_ref, ea_ref):
    s = jnp.sum(p_ref[...], axis=0)
    cm = s[:, 0:5] / N_NODE
    cv = s[:, 8:13] / N_NODE - cm * cm
    cs_ref[...] = jnp.concatenate([cm, cv], axis=0)
    vm = s[:, 16:35] / N_NODE
    vv = s[:, 40:59] / N_NODE - vm * vm
    vs_ref[...] = jnp.concatenate([vm, vv], axis=0)
    em = s[0, 64] / N_EDGE
    ev = s[0, 72] / N_EDGE - em * em
    a = eg_ref[0, 0] / jnp.sqrt(ev + EPS)
    b = eb_ref[0, 0] - em * a
    ea_ref[...] = jnp.concatenate(
        [jnp.full((1, 16), a, f32), jnp.full((1, 16), b, f32)], axis=0)


def _tc_instats(cx, vx, ef128, eg, eb):
    row = lambda i: (i, 0)
    part = pl.pallas_call(
        _instats_part_body,
        grid=(NBLK,),
        in_specs=[pl.BlockSpec((BLK, 5), row), pl.BlockSpec((BLK, 19), row),
                  pl.BlockSpec((N_EDGE // 128, 128), lambda i: (0, 0))],
        out_shape=jax.ShapeDtypeStruct((NBLK, 1, 128), f32),
        out_specs=pl.BlockSpec((1, 1, 128), lambda i: (i, 0, 0)),
    )(cx, vx, ef128)
    return pl.pallas_call(
        _instats_fin_body,
        out_shape=(jax.ShapeDtypeStruct((2, 5), f32),
                   jax.ShapeDtypeStruct((2, 19), f32),
                   jax.ShapeDtypeStruct((2, 16), f32)),
    )(part, eg.reshape(1, 1), eb.reshape(1, 1))


def _embed_body(nproj, x_ref, st_ref, g_ref, b_ref, w1_ref, b1_ref,
                w2_ref, b2_ref, *rest):
    proj_refs = rest[:2 * nproj]
    out_refs = rest[2 * nproj:]
    st = st_ref[...]
    m, v = st[0:1, :], st[1:2, :]
    sc = g_ref[...] / jnp.sqrt(v + EPS)
    xb = (x_ref[...] - m) * sc + (b_ref[...] - 0.0)
    h = jnp.maximum(jnp.dot(xb, w1_ref[...], preferred_element_type=f32)
                    + b1_ref[...], 0.0)
    h = jnp.maximum(jnp.dot(h, w2_ref[...], preferred_element_type=f32)
                    + b2_ref[...], 0.0)
    out_refs[0][...] = h
    for k in range(nproj):
        wt, bt = proj_refs[2 * k], proj_refs[2 * k + 1]
        p = jnp.dot(h, wt[...], preferred_element_type=f32) + bt[...]
        out_refs[1 + 2 * k][...] = p[:, :32]
        out_refs[2 + 2 * k][...] = p[:, 32:]


def _tc_embed(x, st, g, b, w1, b1, w2, b2, projs):
    d = x.shape[1]
    nproj = len(projs)
    row = lambda i: (i, 0)
    zero = lambda i: (0, 0)
    in_specs = ([pl.BlockSpec((BLK, d), row)]
                + [pl.BlockSpec(s.shape, zero) for s in
                   (st, g, b, w1, b1, w2, b2)]
                + [pl.BlockSpec((EMB, EMB), zero), pl.BlockSpec((1, EMB), zero)]
                * nproj)
    out_shape = ([jax.ShapeDtypeStruct((N_NODE, EMB), f32)]
                 + [jax.ShapeDtypeStruct((N_NODE, 32), f32)] * (2 * nproj))
    out_specs = ([pl.BlockSpec((BLK, EMB), row)]
                 + [pl.BlockSpec((BLK, 32), row)] * (2 * nproj))
    args = [x, st, g, b, w1, b1, w2, b2]
    for wt, bt in projs:
        args += [wt, bt]
    return pl.pallas_call(
        functools.partial(_embed_body, nproj),
        grid=(NBLK,), in_specs=in_specs,
        out_shape=tuple(out_shape), out_specs=tuple(out_specs),
    )(*args)


def _bnfin_body(p_ref, g_ref, b_ref, out_ref):
    p = p_ref[...]
    s32 = jnp.sum(p[:16, :32], axis=0, keepdims=True)
    s64 = jnp.sum(p[16:, :32], axis=0, keepdims=True)
    q32 = jnp.sum(p[:16, 32:], axis=0, keepdims=True)
    q64 = jnp.sum(p[16:, 32:], axis=0, keepdims=True)
    s = jnp.concatenate([s32, s64], axis=1)
    q = jnp.concatenate([q32, q64], axis=1)
    m = s / N_EDGE
    v = q / N_EDGE - m * m
    a = g_ref[...] / jnp.sqrt(v + EPS)
    c = b_ref[...] - m * a
    out_ref[...] = jnp.concatenate([a, c], axis=0)


def _tc_bnfin(partials, g, b):
    return pl.pallas_call(
        _bnfin_body,
        out_shape=jax.ShapeDtypeStruct((2, EMB), f32),
    )(partials, g.reshape(1, EMB), b.reshape(1, EMB))


def _convpre_body(s0_ref, s1_ref, cnt_ref, fw_ref, fb_ref,
                  conv_ref, ps_ref):
    S = jnp.concatenate([s0_ref[...], s1_ref[...]], axis=1)
    conv = (jnp.dot(S, fw_ref[...], preferred_element_type=f32)
            + jnp.dot(cnt_ref[...], fb_ref[...], preferred_element_type=f32))
    conv_ref[...] = conv
    ps = jnp.sum(conv, axis=0, keepdims=True)
    pq = jnp.sum(conv * conv, axis=0, keepdims=True)
    ps_ref[...] = jnp.concatenate([ps, pq], axis=1)[None]


def _tc_convpre(s0, s1, cnt, fw, fb):
    row = lambda i: (i, 0)
    zero = lambda i: (0, 0)
    return pl.pallas_call(
        _convpre_body,
        grid=(NBLK,),
        in_specs=[pl.BlockSpec((BLK, 32), row), pl.BlockSpec((BLK, 32), row),
                  pl.BlockSpec((BLK, 32), row),
                  pl.BlockSpec((EMB, EMB), zero), pl.BlockSpec((32, EMB), zero)],
        out_shape=(jax.ShapeDtypeStruct((N_NODE, EMB), f32),
                   jax.ShapeDtypeStruct((NBLK, 1, 128), f32)),
        out_specs=(pl.BlockSpec((BLK, EMB), row),
                   pl.BlockSpec((1, 1, 128), lambda i: (i, 0, 0))),
    )(s0, s1, cnt, fw, jnp.zeros((32, EMB), f32).at[0].set(fb))


def _idxsplit_body(idx_ref, *out_refs):
    idx = idx_ref[...]
    for q in range(4):
        loc = idx - q * QTR
        out_refs[q][...] = jnp.where((loc >= 0) & (loc < QTR), loc, QTR)


def _tc_idxsplit(idxd):
    row = lambda i: (i, 0)
    return pl.pallas_call(
        _idxsplit_body,
        grid=(10,),
        in_specs=[pl.BlockSpec((EROWS // 10, ROWW), row)],
        out_shape=tuple(jax.ShapeDtypeStruct((EROWS, ROWW), i32)
                        for _ in range(4)),
        out_specs=tuple(pl.BlockSpec((EROWS // 10, ROWW), row)
                        for _ in range(4)),
    )(idxd)


def _post_common(conv_ref, ps_ref, prev_ref, pg_ref, pb_ref,
                 o1w_ref, o1b_ref, o2w_ref, o2b_ref):
    ps = jnp.sum(ps_ref[...], axis=0)
    m = ps[:, :EMB] / N_NODE
    v = ps[:, EMB:] / N_NODE - m * m
    a = pg_ref[...] / jnp.sqrt(v + EPS)
    c = pb_ref[...] - m * a
    bnc = conv_ref[...] * a + c
    cat = jnp.concatenate([bnc, prev_ref[...]], axis=1)
    h = jnp.maximum(jnp.dot(cat, o1w_ref[...], preferred_element_type=f32)
                    + o1b_ref[...], 0.0)
    return jnp.maximum(jnp.dot(h, o2w_ref[...], preferred_element_type=f32)
                       + o2b_ref[...], 0.0)


def _post1_body(conv_ref, ps_ref, prev_ref, pg_ref, pb_ref, o1w_ref, o1b_ref,
                o2w_ref, o2b_ref, wl_ref, bl_ref, la_ref, lb_ref):
    y = _post_common(conv_ref, ps_ref, prev_ref, pg_ref, pb_ref,
                     o1w_ref, o1b_ref, o2w_ref, o2b_ref)
    L = jnp.dot(y, wl_ref[...], preferred_element_type=f32) + bl_ref[...]
    la_ref[...] = L[:, :32]
    lb_ref[...] = L[:, 32:]


def _post2_body(conv_ref, ps_ref, prev_ref, pg_ref, pb_ref, o1w_ref, o1b_ref,
                o2w_ref, o2b_ref, w1_ref, b1_ref, w2_ref, b2_ref, out_ref):
    y = _post_common(conv_ref, ps_ref, prev_ref, pg_ref, pb_ref,
                     o1w_ref, o1b_ref, o2w_ref, o2b_ref)
    z = jnp.maximum(jnp.dot(y, w1_ref[...], preferred_element_type=f32)
                    + b1_ref[...], 0.0)
    out_ref[...] = (jnp.dot(z * w2_ref[...], jnp.ones((EMB, 16), f32),
                            preferred_element_type=f32) + b2_ref[...])


def _tc_convpost(body, conv, pstats, prev, pg, pb, o1w, o1b, o2w, o2b,
                 extra, out_shapes, out_specs):
    row = lambda i: (i, 0)
    zero = lambda i: (0, 0)
    small = [pstats, pg.reshape(1, EMB), pb.reshape(1, EMB), o1w,
             o1b.reshape(1, EMB), o2w, o2b.reshape(1, EMB)] + extra
    in_specs = ([pl.BlockSpec((BLK, EMB), row),
                 pl.BlockSpec(pstats.shape, lambda i: (0, 0, 0)),
                 pl.BlockSpec((BLK, EMB), row)]
                + [pl.BlockSpec(a.shape, zero) for a in small[1:]])
    args = [conv, pstats, prev] + small[1:]
    return pl.pallas_call(
        body, grid=(NBLK,), in_specs=in_specs,
        out_shape=out_shapes, out_specs=out_specs,
    )(*args)


_GDN = lax.GatherDimensionNumbers(offset_dims=(), collapsed_slice_dims=(0,),
                                  start_index_map=(0,))
NG = CE // 16 + 1


def _splat(vec16, e_local):
    idx = jnp.full((16, 1), e_local, i32)
    return lax.gather(vec16, idx, dimension_numbers=_GDN, slice_sizes=(1,),
                      mode=lax.GatherScatterMode.PROMISE_IN_BOUNDS)


def _gather_chunk(base, idxl_hbm, idxr_hbm, ef_hbm, l_hbm, r_hbm,
                  idxl_v, idxr_v, ef_v, lrows, rrows, sem):
    pre = [pltpu.async_copy(idxl_hbm.at[pl.ds(base, CHUNK_ROWS)], idxl_v, sem),
           pltpu.async_copy(idxr_hbm.at[pl.ds(base, CHUNK_ROWS)], idxr_v, sem),
           pltpu.async_copy(ef_hbm.at[pl.ds(base * ROWW, CE)], ef_v, sem)]
    for h in pre:
        h.wait()
    handles = []
    for j in range(CHUNK_ROWS):
        handles.append(pltpu.async_copy(
            l_hbm.at[idxl_v.at[j]], lrows.at[pl.ds(j * ROWW, ROWW)], sem))
        handles.append(pltpu.async_copy(
            r_hbm.at[idxr_v.at[j]], rrows.at[pl.ds(j * ROWW, ROWW)], sem))
    for h in handles:
        h.wait()


def _sc_stats_fn(idxl_hbm, idxr_hbm, ef_hbm, la_hbm, lb_hbm, ra_hbm, rb_hbm,
                 wea_hbm, web_hbm, ea_hbm, out_hbm,
                 idxl_v, idxr_v, ef_v, lrows, rrows, we_v, ea_v, st_v, sem):
    c = lax.axis_index("c")
    s = lax.axis_index("s")
    pltpu.sync_copy(ea_hbm, ea_v)

    def run(l_hbm, r_hbm, we_hbm):
        pltpu.sync_copy(we_hbm, we_v)
        a16 = ea_v[pl.ds(0, 16)]
        b16 = ea_v[pl.ds(16, 16)]
        we0 = we_v[pl.ds(0, 16)]
        we1 = we_v[pl.ds(16, 16)]

        def chunk(i, acc):
            base = s * ROWS_PER_TILE + i * CHUNK_ROWS
            _gather_chunk(base, idxl_hbm, idxr_hbm, ef_hbm, l_hbm, r_hbm,
                          idxl_v, idxr_v, ef_v, lrows, rrows, sem)

            def group(g, acc2):
                s0, s1, q0, q1 = acc2
                gb = jnp.minimum(g * 16, CE - 16)
                ef16 = ef_v[pl.ds(gb, 16)]
                vd = jnp.where(g < NG - 1, 1.0, 0.0).astype(f32)
                for el in range(16):
                    e = gb + el
                    efn = _splat(ef16, el) * a16 + b16
                    j0 = efn * we0 + lrows[e, pl.ds(0, 16)] + rrows[e, pl.ds(0, 16)]
                    j1 = efn * we1 + lrows[e, pl.ds(16, 16)] + rrows[e, pl.ds(16, 16)]
                    if el < 8:
                        s0 = s0 + j0 * vd
                        s1 = s1 + j1 * vd
                        q0 = q0 + (j0 * j0) * vd
                        q1 = q1 + (j1 * j1) * vd
                    else:
                        s0 = s0 + j0
                        s1 = s1 + j1
                        q0 = q0 + j0 * j0
                        q1 = q1 + j1 * j1
                return (s0, s1, q0, q1)
            zc = jnp.zeros((16,), f32)
            cs0, cs1, cq0, cq1 = lax.fori_loop(0, NG, group, (zc, zc, zc, zc))
            return (acc[0] + cs0, acc[1] + cs1, acc[2] + cq0, acc[3] + cq1)

        z = jnp.zeros((16,), f32)
        s0, s1, q0, q1 = lax.fori_loop(0, N_CHUNK, chunk, (z, z, z, z))
        st_v[pl.ds(0, 16)] = s0
        st_v[pl.ds(16, 16)] = s1
        st_v[pl.ds(32, 16)] = q0
        st_v[pl.ds(48, 16)] = q1
        pltpu.sync_copy(st_v, out_hbm.at[c * N_TILE + s])

    @pl.when(c == 0)
    def _():
        run(la_hbm, ra_hbm, wea_hbm)

    @pl.when(c == 1)
    def _():
        run(lb_hbm, rb_hbm, web_hbm)


def _sc_stats(idxl, idxr, ef1, la, lb, ra, rb, wea, web, ea):
    kfn = functools.partial(
        pl.kernel, mesh=_mesh,
        compiler_params=pltpu.CompilerParams(use_tc_tiling_on_sc=False),
        out_type=jax.ShapeDtypeStruct((32, 64), f32),
        scratch_types=[pltpu.VMEM((CHUNK_ROWS, ROWW), i32),
                       pltpu.VMEM((CHUNK_ROWS, ROWW), i32),
                       pltpu.VMEM((CE,), f32),
                       pltpu.VMEM((CE, 32), f32),
                       pltpu.VMEM((CE, 32), f32),
                       pltpu.VMEM((32,), f32),
                       pltpu.VMEM((32,), f32),
                       pltpu.VMEM((64,), f32),
                       pltpu.SemaphoreType.DMA],
    )
    return kfn(_sc_stats_fn)(idxl, idxr, ef1, la, lb, ra, rb, wea, web, ea)


def _sc_hcompute_fn(idxl_hbm, idxr_hbm, ef_hbm,
                    la_hbm, lb_hbm, ra_hbm, rb_hbm, wea_hbm, web_hbm, ea_hbm,
                    affa_hbm, affb_hbm, h0_hbm, h1_hbm,
                    idxl_v, idxr_v, ef_v, lrows, rrows,
                    we_v, ea_v, aff_v, hst, sem):
    c = lax.axis_index("c")
    s = lax.axis_index("s")
    pltpu.sync_copy(ea_hbm, ea_v)

    def run(l_hbm, r_hbm, we_hbm, aff_hbm, h_hbm):
        pltpu.sync_copy(we_hbm, we_v)
        pltpu.sync_copy(aff_hbm, aff_v)
        a16 = ea_v[pl.ds(0, 16)]
        b16 = ea_v[pl.ds(16, 16)]
        we0 = we_v[pl.ds(0, 16)]
        we1 = we_v[pl.ds(16, 16)]
        aa0 = aff_v[pl.ds(0, 16)]
        aa1 = aff_v[pl.ds(16, 16)]
        ac0 = aff_v[pl.ds(32, 16)]
        ac1 = aff_v[pl.ds(48, 16)]

        def chunk(i, _):
            base = s * ROWS_PER_TILE + i * CHUNK_ROWS
            _gather_chunk(base, idxl_hbm, idxr_hbm, ef_hbm, l_hbm, r_hbm,
                          idxl_v, idxr_v, ef_v, lrows, rrows, sem)

            def group(g, _2):
                gb = jnp.minimum(g * 16, CE - 16)
                ef16 = ef_v[pl.ds(gb, 16)]
                for el in range(16):
                    e = gb + el
                    efn = _splat(ef16, el) * a16 + b16
                    j0 = (efn * we0 + lrows[e, pl.ds(0, 16)]
                          + rrows[e, pl.ds(0, 16)])
                    j1 = (efn * we1 + lrows[e, pl.ds(16, 16)]
                          + rrows[e, pl.ds(16, 16)])
                    hst[e, pl.ds(0, 16)] = jnp.maximum(j0 * aa0 + ac0, 0.0)
                    hst[e, pl.ds(16, 16)] = jnp.maximum(j1 * aa1 + ac1, 0.0)
                return 0
            lax.fori_loop(0, NG, group, 0)
            pltpu.sync_copy(hst, h_hbm.at[pl.ds(base * ROWW, CE)])
            return 0
        lax.fori_loop(0, N_CHUNK, chunk, 0)

    @pl.when(c == 0)
    def _():
        run(la_hbm, ra_hbm, wea_hbm, affa_hbm, h0_hbm)

    @pl.when(c == 1)
    def _():
        run(lb_hbm, rb_hbm, web_hbm, affb_hbm, h1_hbm)


def _sc_hcompute(idxl, idxr, ef1, la, lb, ra, rb, wea, web, ea, affa, affb):
    kfn = functools.partial(
        pl.kernel, mesh=_mesh,
        compiler_params=pltpu.CompilerParams(use_tc_tiling_on_sc=False),
        out_type=(jax.ShapeDtypeStruct((N_EDGE, 32), f32),
                  jax.ShapeDtypeStruct((N_EDGE, 32), f32)),
        scratch_types=[pltpu.VMEM((CHUNK_ROWS, ROWW), i32),
                       pltpu.VMEM((CHUNK_ROWS, ROWW), i32),
                       pltpu.VMEM((CE,), f32),
                       pltpu.VMEM((CE, 32), f32),
                       pltpu.VMEM((CE, 32), f32),
                       pltpu.VMEM((32,), f32),
                       pltpu.VMEM((32,), f32),
                       pltpu.VMEM((64,), f32),
                       pltpu.VMEM((CE, 32), f32),
                       pltpu.SemaphoreType.DMA],
    )
    return kfn(_sc_hcompute_fn)(idxl, idxr, ef1, la, lb, ra, rb,
                                wea, web, ea, affa, affb)


def _sc_scatter_fn(idq0_hbm, idq1_hbm, idq2_hbm, idq3_hbm,
                   h0_hbm, h1_hbm, z_hbm, ones_hbm,
                   s0_hbm, s1_hbm, cnt_hbm,
                   idxd_v, hst, ones_v, acc_sh, sem):
    c = lax.axis_index("c")
    s = lax.axis_index("s")
    idqs = (idq0_hbm, idq1_hbm, idq2_hbm, idq3_hbm)
    pltpu.sync_copy(ones_hbm, ones_v)

    def run(h_hbm, out_hbm, cnt_q):
        for quarter in range(4):
            idxd_hbm = idqs[quarter]
            zoff = s * (QPAD // N_TILE)
            pltpu.sync_copy(z_hbm.at[pl.ds(zoff, QPAD // N_TILE)],
                            acc_sh.at[pl.ds(zoff, QPAD // N_TILE)])
            plsc.subcore_barrier()

            def chunk(i, _):
                base = s * ROWS_PER_TILE + i * CHUNK_ROWS
                pre = [pltpu.async_copy(idxd_hbm.at[pl.ds(base, CHUNK_ROWS)],
                                        idxd_v, sem),
                       pltpu.async_copy(h_hbm.at[pl.ds(base * ROWW, CE)],
                                        hst, sem)]
                for h in pre:
                    h.wait()
                sc = [pltpu.async_copy(hst.at[pl.ds(j * ROWW, ROWW)],
                                       acc_sh.at[idxd_v.at[j]], sem, add=True)
                      for j in range(CHUNK_ROWS)]
                for h in sc:
                    h.wait()
                return 0
            lax.fori_loop(0, N_CHUNK, chunk, 0)
            plsc.subcore_barrier()

            @pl.when(s < 4)
            def _():
                coff = s * (QTR // 4)
                pltpu.sync_copy(
                    acc_sh.at[pl.ds(coff, QTR // 4)],
                    out_hbm.at[pl.ds(quarter * QTR + coff, QTR // 4)])
            plsc.subcore_barrier()

        for qq in range(2):
            zoff = s * (QPAD // N_TILE)
            pltpu.sync_copy(z_hbm.at[pl.ds(zoff, QPAD // N_TILE)],
                            acc_sh.at[pl.ds(zoff, QPAD // N_TILE)])
            plsc.subcore_barrier()
            idc_hbm = idqs[cnt_q + qq]

            def cchunk(i, _):
                base = s * ROWS_PER_TILE + i * CHUNK_ROWS
                pltpu.sync_copy(idc_hbm.at[pl.ds(base, CHUNK_ROWS)], idxd_v)
                sc = [pltpu.async_copy(ones_v, acc_sh.at[idxd_v.at[j]],
                                       sem, add=True)
                      for j in range(CHUNK_ROWS)]
                for h in sc:
                    h.wait()
                return 0
            lax.fori_loop(0, N_CHUNK, cchunk, 0)
            plsc.subcore_barrier()

            @pl.when(s < 4)
            def _():
                coff = s * (QTR // 4)
                pltpu.sync_copy(
                    acc_sh.at[pl.ds(coff, QTR // 4)],
                    cnt_hbm.at[pl.ds((cnt_q + qq) * QTR + coff, QTR // 4)])
            plsc.subcore_barrier()

    @pl.when(c == 0)
    def _():
        run(h0_hbm, s0_hbm, 0)

    @pl.when(c == 1)
    def _():
        run(h1_hbm, s1_hbm, 2)


def _sc_scatter(idq, h0, h1, zh, ones32):
    kfn = functools.partial(
        pl.kernel, mesh=_mesh,
        compiler_params=pltpu.CompilerParams(use_tc_tiling_on_sc=False),
        out_type=(jax.ShapeDtypeStruct((N_NODE, 32), f32),
                  jax.ShapeDtypeStruct((N_NODE, 32), f32),
                  jax.ShapeDtypeStruct((N_NODE, 32), f32)),
        scratch_types=[pltpu.VMEM((CHUNK_ROWS, ROWW), i32),
                       pltpu.VMEM((CE, 32), f32),
                       pltpu.VMEM((ROWW, 32), f32),
                       pltpu.VMEM_SHARED((QPAD, 32), f32),
                       pltpu.SemaphoreType.DMA],
    )
    return kfn(_sc_scatter_fn)(idq[0], idq[1], idq[2], idq[3],
                               h0, h1, zh, ones32)


def _sc_cand_fn(log_hbm, cand_hbm, out_hbm, cidx_v, rows_v, sem):
    c = lax.axis_index("c")
    s = lax.axis_index("s")
    w = s * 2 + c
    pltpu.sync_copy(cand_hbm.at[pl.ds(w * 128, 128)], cidx_v)
    pltpu.async_copy(log_hbm.at[cidx_v], rows_v, sem).wait()
    pltpu.sync_copy(rows_v, out_hbm.at[pl.ds(w * 128, 128)])


def _sc_cand(logits16, cand):
    kfn = functools.partial(
        pl.kernel, mesh=_mesh,
        compiler_params=pltpu.CompilerParams(use_tc_tiling_on_sc=False),
        out_type=jax.ShapeDtypeStruct((4096, 16), f32),
        scratch_types=[pltpu.VMEM((128,), i32),
                       pltpu.VMEM((128, 16), f32),
                       pltpu.SemaphoreType.DMA],
    )
    return kfn(_sc_cand_fn)(logits16, cand)


def kernel(constraint_features, edge_indices, edge_features, variable_features,
           candidates, constraints_per_sample, variables_per_sample,
           candidates_per_sample, params):
    p = params
    idx0 = edge_indices[0].reshape(EROWS, ROWW)
    idx1 = edge_indices[1].reshape(EROWS, ROWW)
    ef1 = edge_features.reshape(N_EDGE)

    cs, vs, ea = _tc_instats(constraint_features, variable_features,
                             edge_features.reshape(N_EDGE // 128, 128),
                             p['edge_bn_g'], p['edge_bn_b'])
    ea16 = ea[0]
    eb16 = ea[1]
    ea2 = jnp.concatenate([ea16, eb16])

    cf0, l1a, l1b = _tc_embed(
        constraint_features, cs, p['cons_bn_g'].reshape(1, 5),
        p['cons_bn_b'].reshape(1, 5), p['cons_W1'].T,
        p['cons_b1'].reshape(1, EMB), p['cons_W2'].T,
        p['cons_b2'].reshape(1, EMB),
        [(p['vc_Wl'].T, p['vc_bl'].reshape(1, EMB))])
    zerob = jnp.zeros((1, EMB), f32)
    vf0, r1a, r1b, r2a, r2b = _tc_embed(
        variable_features, vs, p['var_bn_g'].reshape(1, 19),
        p['var_bn_b'].reshape(1, 19), p['var_W1'].T,
        p['var_b1'].reshape(1, EMB), p['var_W2'].T,
        p['var_b2'].reshape(1, EMB),
        [(p['vc_Wr'].T, zerob), (p['cv_Wr'].T, zerob)])

    ones32 = jnp.ones((ROWW, 32), f32)
    zh = jnp.zeros((QPAD, 32), f32)

    def conv_pass(la, lb, ra, rb, we, fg, fb, idxd, finW, finb):
        wea, web = we[:32, 0], we[32:, 0]
        part = _sc_stats(idx0, idx1, ef1, la, lb, ra, rb, wea, web, ea2)
        aff = _tc_bnfin(part, fg, fb)
        affa = jnp.concatenate([aff[0, :32], aff[1, :32]])
        affb = jnp.concatenate([aff[0, 32:], aff[1, 32:]])
        idq = _tc_idxsplit(idxd)
        h0, h1 = _sc_hcompute(idx0, idx1, ef1, la, lb, ra, rb,
                              wea, web, ea2, affa, affb)
        s0, s1, cnt = _sc_scatter(idq, h0, h1, zh, ones32)
        return _tc_convpre(s0, s1, cnt, finW.T, finb)

    conv1, ps1 = conv_pass(l1a, l1b, r1a, r1b, p['vc_We'],
                           p['vc_fin_bn_g'], p['vc_fin_bn_b'],
                           idx0, p['vc_fin_W'], p['vc_fin_b'])
    row = lambda i: (i, 0)
    l2a, l2b = _tc_convpost(
        _post1_body, conv1, ps1, cf0, p['vc_post_bn_g'], p['vc_post_bn_b'],
        p['vc_o1_W'].T, p['vc_o1_b'], p['vc_o2_W'].T, p['vc_o2_b'],
        [p['cv_Wl'].T, p['cv_bl'].reshape(1, EMB)],
        (jax.ShapeDtypeStruct((N_NODE, 32), f32),
         jax.ShapeDtypeStruct((N_NODE, 32), f32)),
        (pl.BlockSpec((BLK, 32), row), pl.BlockSpec((BLK, 32), row)))

    conv2, ps2 = conv_pass(l2a, l2b, r2a, r2b, p['cv_We'],
                           p['cv_fin_bn_g'], p['cv_fin_bn_b'],
                           idx1, p['cv_fin_W'], p['cv_fin_b'])
    logits16 = _tc_convpost(
        _post2_body, conv2, ps2, vf0, p['cv_post_bn_g'], p['cv_post_bn_b'],
        p['cv_o1_W'].T, p['cv_o1_b'], p['cv_o2_W'].T, p['cv_o2_b'],
        [p['out_W1'].T, p['out_b1'].reshape(1, EMB),
         p['out_W2'].reshape(1, EMB), p['out_b2'].reshape(1, 1)],
        jax.ShapeDtypeStruct((N_NODE, 16), f32),
        pl.BlockSpec((BLK, 16), row))

    out = _sc_cand(logits16, candidates)
    return out[:, 0:1]


# speedup vs baseline: 1.3935x; 1.3935x over previous
import functools

import jax
import jax.numpy as jnp
from jax import lax
from jax.experimental import pallas as pl
from jax.experimental.pallas import tpu as pltpu
from jax.experimental.pallas import tpu_sc as plsc

EMB = 64
N_NODE = 50000
N_EDGE = 800000
ROWW = 125
EROWS = N_EDGE // ROWW
N_TILE = 16
ROWS_PER_TILE = EROWS // N_TILE
CHUNK_ROWS = 8
N_CHUNK = ROWS_PER_TILE // CHUNK_ROWS
CE = CHUNK_ROWS * ROWW
NBLK = 10
BLK = N_NODE // NBLK
EPS = 1e-5
HALF = N_NODE // 2
HPAD = HALF + 8

_mesh = plsc.VectorSubcoreMesh(core_axis_name="c", subcore_axis_name="s")
f32 = jnp.float32
i32 = jnp.int32


def _instats_part_body(cx_ref, vx_ref, ef_ref, out_ref):
    cx = cx_ref[...]
    vx = vx_ref[...]
    ef = ef_ref[...]
    z = lambda n: jnp.zeros((1, n), f32)
    w = jnp.where(pl.program_id(0) == 0, 1.0, 0.0).astype(f32)
    out_ref[...] = jnp.concatenate([
        jnp.sum(cx, axis=0, keepdims=True), z(3),
        jnp.sum(cx * cx, axis=0, keepdims=True), z(3),
        jnp.sum(vx, axis=0, keepdims=True), z(5),
        jnp.sum(vx * vx, axis=0, keepdims=True), z(5),
        (w * jnp.sum(ef)).reshape(1, 1), z(7),
        (w * jnp.sum(ef * ef)).reshape(1, 1), z(55)], axis=1)[None]


def _instats_fin_body(p_ref, eg_ref, eb_ref, cs_ref, vs_ref, ea_ref):
    s = jnp.sum(p_ref[...], axis=0)
    cm = s[:, 0:5] / N_NODE
    cv = s[:, 8:13] / N_NODE - cm * cm
    cs_ref[...] = jnp.concatenate([cm, cv], axis=0)
    vm = s[:, 16:35] / N_NODE
    vv = s[:, 40:59] / N_NODE - vm * vm
    vs_ref[...] = jnp.concatenate([vm, vv], axis=0)
    em = s[0, 64] / N_EDGE
    ev = s[0, 72] / N_EDGE - em * em
    a = eg_ref[0, 0] / jnp.sqrt(ev + EPS)
    b = eb_ref[0, 0] - em * a
    ea_ref[...] = jnp.concatenate(
        [jnp.full((1, 16), a, f32), jnp.full((1, 16), b, f32)], axis=0)


def _tc_instats(cx, vx, ef128, eg, eb):
    row = lambda i: (i, 0)
    part = pl.pallas_call(
        _instats_part_body,
        grid=(NBLK,),
        in_specs=[pl.BlockSpec((BLK, 5), row), pl.BlockSpec((BLK, 19), row),
                  pl.BlockSpec((N_EDGE // 128, 128), lambda i: (0, 0))],
        out_shape=jax.ShapeDtypeStruct((NBLK, 1, 128), f32),
        out_specs=pl.BlockSpec((1, 1, 128), lambda i: (i, 0, 0)),
    )(cx, vx, ef128)
    return pl.pallas_call(
        _instats_fin_body,
        out_shape=(jax.ShapeDtypeStruct((2, 5), f32),
                   jax.ShapeDtypeStruct((2, 19), f32),
                   jax.ShapeDtypeStruct((2, 16), f32)),
    )(part, eg.reshape(1, 1), eb.reshape(1, 1))


def _embed_body(nproj, x_ref, st_ref, g_ref, b_ref, w1_ref, b1_ref,
                w2_ref, b2_ref, *rest):
    proj_refs = rest[:2 * nproj]
    out_refs = rest[2 * nproj:]
    st = st_ref[...]
    m, v = st[0:1, :], st[1:2, :]
    sc = g_ref[...] / jnp.sqrt(v + EPS)
    xb = (x_ref[...] - m) * sc + (b_ref[...] - 0.0)
    h = jnp.maximum(jnp.dot(xb, w1_ref[...], preferred_element_type=f32)
                    + b1_ref[...], 0.0)
    h = jnp.maximum(jnp.dot(h, w2_ref[...], preferred_element_type=f32)
                    + b2_ref[...], 0.0)
    out_refs[0][...] = h
    for k in range(nproj):
        wt, bt = proj_refs[2 * k], proj_refs[2 * k + 1]
        p = jnp.dot(h, wt[...], preferred_element_type=f32) + bt[...]
        out_refs[1 + 2 * k][...] = p[:, :32]
        out_refs[2 + 2 * k][...] = p[:, 32:]


def _tc_embed(x, st, g, b, w1, b1, w2, b2, projs):
    d = x.shape[1]
    nproj = len(projs)
    row = lambda i: (i, 0)
    zero = lambda i: (0, 0)
    in_specs = ([pl.BlockSpec((BLK, d), row)]
                + [pl.BlockSpec(s.shape, zero) for s in
                   (st, g, b, w1, b1, w2, b2)]
                + [pl.BlockSpec((EMB, EMB), zero), pl.BlockSpec((1, EMB), zero)]
                * nproj)
    out_shape = ([jax.ShapeDtypeStruct((N_NODE, EMB), f32)]
                 + [jax.ShapeDtypeStruct((N_NODE, 32), f32)] * (2 * nproj))
    out_specs = ([pl.BlockSpec((BLK, EMB), row)]
                 + [pl.BlockSpec((BLK, 32), row)] * (2 * nproj))
    args = [x, st, g, b, w1, b1, w2, b2]
    for wt, bt in projs:
        args += [wt, bt]
    return pl.pallas_call(
        functools.partial(_embed_body, nproj),
        grid=(NBLK,), in_specs=in_specs,
        out_shape=tuple(out_shape), out_specs=tuple(out_specs),
    )(*args)


def _bnfin_body(p_ref, g_ref, b_ref, out_ref):
    p = p_ref[...]
    s32 = jnp.sum(p[:16, :32], axis=0, keepdims=True)
    s64 = jnp.sum(p[16:, :32], axis=0, keepdims=True)
    q32 = jnp.sum(p[:16, 32:], axis=0, keepdims=True)
    q64 = jnp.sum(p[16:, 32:], axis=0, keepdims=True)
    s = jnp.concatenate([s32, s64], axis=1)
    q = jnp.concatenate([q32, q64], axis=1)
    m = s / N_EDGE
    v = q / N_EDGE - m * m
    a = g_ref[...] / jnp.sqrt(v + EPS)
    c = b_ref[...] - m * a
    out_ref[...] = jnp.concatenate([a, c], axis=0)


def _tc_bnfin(partials, g, b):
    return pl.pallas_call(
        _bnfin_body,
        out_shape=jax.ShapeDtypeStruct((2, EMB), f32),
    )(partials, g.reshape(1, EMB), b.reshape(1, EMB))


def _convpre_body(s0_ref, s1_ref, s2_ref, s3_ref, cnt_ref, fw_ref, fb_ref,
                  conv_ref, ps_ref):
    S = jnp.concatenate([s0_ref[...], s1_ref[...], s2_ref[...], s3_ref[...]],
                        axis=1)
    conv = (jnp.dot(S, fw_ref[...], preferred_element_type=f32)
            + jnp.dot(cnt_ref[...], fb_ref[...], preferred_element_type=f32))
    conv_ref[...] = conv
    ps = jnp.sum(conv, axis=0, keepdims=True)
    pq = jnp.sum(conv * conv, axis=0, keepdims=True)
    ps_ref[...] = jnp.concatenate([ps, pq], axis=1)[None]


def _tc_convpre(s4, cnt, fw, fb):
    row = lambda i: (i, 0)
    zero = lambda i: (0, 0)
    return pl.pallas_call(
        _convpre_body,
        grid=(NBLK,),
        in_specs=[pl.BlockSpec((BLK, 16), row)] * 5
        + [pl.BlockSpec((EMB, EMB), zero), pl.BlockSpec((16, EMB), zero)],
        out_shape=(jax.ShapeDtypeStruct((N_NODE, EMB), f32),
                   jax.ShapeDtypeStruct((NBLK, 1, 128), f32)),
        out_specs=(pl.BlockSpec((BLK, EMB), row),
                   pl.BlockSpec((1, 1, 128), lambda i: (i, 0, 0))),
    )(*s4, cnt, fw, jnp.zeros((16, EMB), f32).at[0].set(fb))


def _idxsplit_body(idx_ref, *out_refs):
    idx = idx_ref[...]
    for q in range(2):
        loc = idx - q * HALF
        out_refs[q][...] = jnp.where((loc >= 0) & (loc < HALF), loc, HALF)


def _tc_idxsplit(idxd):
    row = lambda i: (i, 0)
    return pl.pallas_call(
        _idxsplit_body,
        grid=(10,),
        in_specs=[pl.BlockSpec((EROWS // 10, ROWW), row)],
        out_shape=tuple(jax.ShapeDtypeStruct((EROWS, ROWW), i32)
                        for _ in range(2)),
        out_specs=tuple(pl.BlockSpec((EROWS // 10, ROWW), row)
                        for _ in range(2)),
    )(idxd)


def _post_common(conv_ref, ps_ref, prev_ref, pg_ref, pb_ref,
                 o1w_ref, o1b_ref, o2w_ref, o2b_ref):
    ps = jnp.sum(ps_ref[...], axis=0)
    m = ps[:, :EMB] / N_NODE
    v = ps[:, EMB:] / N_NODE - m * m
    a = pg_ref[...] / jnp.sqrt(v + EPS)
    c = pb_ref[...] - m * a
    bnc = conv_ref[...] * a + c
    cat = jnp.concatenate([bnc, prev_ref[...]], axis=1)
    h = jnp.maximum(jnp.dot(cat, o1w_ref[...], preferred_element_type=f32)
                    + o1b_ref[...], 0.0)
    return jnp.maximum(jnp.dot(h, o2w_ref[...], preferred_element_type=f32)
                       + o2b_ref[...], 0.0)


def _post1_body(conv_ref, ps_ref, prev_ref, pg_ref, pb_ref, o1w_ref, o1b_ref,
                o2w_ref, o2b_ref, wl_ref, bl_ref, la_ref, lb_ref):
    y = _post_common(conv_ref, ps_ref, prev_ref, pg_ref, pb_ref,
                     o1w_ref, o1b_ref, o2w_ref, o2b_ref)
    L = jnp.dot(y, wl_ref[...], preferred_element_type=f32) + bl_ref[...]
    la_ref[...] = L[:, :32]
    lb_ref[...] = L[:, 32:]


def _post2_body(conv_ref, ps_ref, prev_ref, pg_ref, pb_ref, o1w_ref, o1b_ref,
                o2w_ref, o2b_ref, w1_ref, b1_ref, w2_ref, b2_ref, out_ref):
    y = _post_common(conv_ref, ps_ref, prev_ref, pg_ref, pb_ref,
                     o1w_ref, o1b_ref, o2w_ref, o2b_ref)
    z = jnp.maximum(jnp.dot(y, w1_ref[...], preferred_element_type=f32)
                    + b1_ref[...], 0.0)
    out_ref[...] = (jnp.dot(z * w2_ref[...], jnp.ones((EMB, 16), f32),
                            preferred_element_type=f32) + b2_ref[...])


def _tc_convpost(body, conv, pstats, prev, pg, pb, o1w, o1b, o2w, o2b,
                 extra, out_shapes, out_specs):
    row = lambda i: (i, 0)
    zero = lambda i: (0, 0)
    small = [pstats, pg.reshape(1, EMB), pb.reshape(1, EMB), o1w,
             o1b.reshape(1, EMB), o2w, o2b.reshape(1, EMB)] + extra
    in_specs = ([pl.BlockSpec((BLK, EMB), row),
                 pl.BlockSpec(pstats.shape, lambda i: (0, 0, 0)),
                 pl.BlockSpec((BLK, EMB), row)]
                + [pl.BlockSpec(a.shape, zero) for a in small[1:]])
    args = [conv, pstats, prev] + small[1:]
    return pl.pallas_call(
        body, grid=(NBLK,), in_specs=in_specs,
        out_shape=out_shapes, out_specs=out_specs,
    )(*args)


_GDN = lax.GatherDimensionNumbers(offset_dims=(), collapsed_slice_dims=(0,),
                                  start_index_map=(0,))
NG = CE // 16 + 1


def _splat(vec16, e_local):
    idx = jnp.full((16, 1), e_local, i32)
    return lax.gather(vec16, idx, dimension_numbers=_GDN, slice_sizes=(1,),
                      mode=lax.GatherScatterMode.PROMISE_IN_BOUNDS)


def _gather_chunk(base, idxl_hbm, idxr_hbm, ef_hbm, l_hbm, r_hbm,
                  idxl_v, idxr_v, ef_v, lrows, rrows, sem):
    pre = [pltpu.async_copy(idxl_hbm.at[pl.ds(base, CHUNK_ROWS)], idxl_v, sem),
           pltpu.async_copy(idxr_hbm.at[pl.ds(base, CHUNK_ROWS)], idxr_v, sem),
           pltpu.async_copy(ef_hbm.at[pl.ds(base * ROWW, CE)], ef_v, sem)]
    for h in pre:
        h.wait()
    handles = []
    for j in range(CHUNK_ROWS):
        handles.append(pltpu.async_copy(
            l_hbm.at[idxl_v.at[j]], lrows.at[pl.ds(j * ROWW, ROWW)], sem))
        handles.append(pltpu.async_copy(
            r_hbm.at[idxr_v.at[j]], rrows.at[pl.ds(j * ROWW, ROWW)], sem))
    for h in handles:
        h.wait()


def _sc_stats_fn(idxl_hbm, idxr_hbm, ef_hbm, la_hbm, lb_hbm, ra_hbm, rb_hbm,
                 wea_hbm, web_hbm, ea_hbm, out_hbm,
                 idxl_v, idxr_v, ef_v, lrows, rrows, we_v, ea_v, st_v, sem):
    c = lax.axis_index("c")
    s = lax.axis_index("s")
    pltpu.sync_copy(ea_hbm, ea_v)

    def run(l_hbm, r_hbm, we_hbm):
        pltpu.sync_copy(we_hbm, we_v)
        a16 = ea_v[pl.ds(0, 16)]
        b16 = ea_v[pl.ds(16, 16)]
        we0 = we_v[pl.ds(0, 16)]
        we1 = we_v[pl.ds(16, 16)]

        def chunk(i, acc):
            base = s * ROWS_PER_TILE + i * CHUNK_ROWS
            _gather_chunk(base, idxl_hbm, idxr_hbm, ef_hbm, l_hbm, r_hbm,
                          idxl_v, idxr_v, ef_v, lrows, rrows, sem)

            def group(g, acc2):
                s0, s1, q0, q1 = acc2
                gb = jnp.minimum(g * 16, CE - 16)
                ef16 = ef_v[pl.ds(gb, 16)]
                vd = jnp.where(g < NG - 1, 1.0, 0.0).astype(f32)
                for el in range(16):
                    e = gb + el
                    efn = _splat(ef16, el) * a16 + b16
                    j0 = efn * we0 + lrows[e, pl.ds(0, 16)] + rrows[e, pl.ds(0, 16)]
                    j1 = efn * we1 + lrows[e, pl.ds(16, 16)] + rrows[e, pl.ds(16, 16)]
                    if el < 8:
                        s0 = s0 + j0 * vd
                        s1 = s1 + j1 * vd
                        q0 = q0 + (j0 * j0) * vd
                        q1 = q1 + (j1 * j1) * vd
                    else:
                        s0 = s0 + j0
                        s1 = s1 + j1
                        q0 = q0 + j0 * j0
                        q1 = q1 + j1 * j1
                return (s0, s1, q0, q1)
            zc = jnp.zeros((16,), f32)
            cs0, cs1, cq0, cq1 = lax.fori_loop(0, NG, group, (zc, zc, zc, zc))
            return (acc[0] + cs0, acc[1] + cs1, acc[2] + cq0, acc[3] + cq1)

        z = jnp.zeros((16,), f32)
        s0, s1, q0, q1 = lax.fori_loop(0, N_CHUNK, chunk, (z, z, z, z))
        st_v[pl.ds(0, 16)] = s0
        st_v[pl.ds(16, 16)] = s1
        st_v[pl.ds(32, 16)] = q0
        st_v[pl.ds(48, 16)] = q1
        pltpu.sync_copy(st_v, out_hbm.at[c * N_TILE + s])

    @pl.when(c == 0)
    def _():
        run(la_hbm, ra_hbm, wea_hbm)

    @pl.when(c == 1)
    def _():
        run(lb_hbm, rb_hbm, web_hbm)


def _sc_stats(idxl, idxr, ef1, la, lb, ra, rb, wea, web, ea):
    kfn = functools.partial(
        pl.kernel, mesh=_mesh,
        compiler_params=pltpu.CompilerParams(use_tc_tiling_on_sc=False),
        out_type=jax.ShapeDtypeStruct((32, 64), f32),
        scratch_types=[pltpu.VMEM((CHUNK_ROWS, ROWW), i32),
                       pltpu.VMEM((CHUNK_ROWS, ROWW), i32),
                       pltpu.VMEM((CE,), f32),
                       pltpu.VMEM((CE, 32), f32),
                       pltpu.VMEM((CE, 32), f32),
                       pltpu.VMEM((32,), f32),
                       pltpu.VMEM((32,), f32),
                       pltpu.VMEM((64,), f32),
                       pltpu.SemaphoreType.DMA],
    )
    return kfn(_sc_stats_fn)(idxl, idxr, ef1, la, lb, ra, rb, wea, web, ea)


def _sc_hcompute_fn(idxl_hbm, idxr_hbm, ef_hbm,
                    la_hbm, lb_hbm, ra_hbm, rb_hbm, wea_hbm, web_hbm, ea_hbm,
                    affa_hbm, affb_hbm, h00_hbm, h01_hbm, h10_hbm, h11_hbm,
                    idxl_v, idxr_v, ef_v, lrows, rrows,
                    we_v, ea_v, aff_v, hst0, hst1, sem):
    c = lax.axis_index("c")
    s = lax.axis_index("s")
    pltpu.sync_copy(ea_hbm, ea_v)

    def run(l_hbm, r_hbm, we_hbm, aff_hbm, h0_hbm, h1_hbm):
        pltpu.sync_copy(we_hbm, we_v)
        pltpu.sync_copy(aff_hbm, aff_v)
        a16 = ea_v[pl.ds(0, 16)]
        b16 = ea_v[pl.ds(16, 16)]
        we0 = we_v[pl.ds(0, 16)]
        we1 = we_v[pl.ds(16, 16)]
        aa0 = aff_v[pl.ds(0, 16)]
        aa1 = aff_v[pl.ds(16, 16)]
        ac0 = aff_v[pl.ds(32, 16)]
        ac1 = aff_v[pl.ds(48, 16)]

        def chunk(i, _):
            base = s * ROWS_PER_TILE + i * CHUNK_ROWS
            _gather_chunk(base, idxl_hbm, idxr_hbm, ef_hbm, l_hbm, r_hbm,
                          idxl_v, idxr_v, ef_v, lrows, rrows, sem)

            def group(g, _2):
                gb = jnp.minimum(g * 16, CE - 16)
                ef16 = ef_v[pl.ds(gb, 16)]
                for el in range(16):
                    e = gb + el
                    efn = _splat(ef16, el) * a16 + b16
                    j0 = (efn * we0 + lrows[e, pl.ds(0, 16)]
                          + rrows[e, pl.ds(0, 16)])
                    j1 = (efn * we1 + lrows[e, pl.ds(16, 16)]
                          + rrows[e, pl.ds(16, 16)])
                    hst0[e, pl.ds(0, 16)] = jnp.maximum(j0 * aa0 + ac0, 0.0)
                    hst1[e, pl.ds(0, 16)] = jnp.maximum(j1 * aa1 + ac1, 0.0)
                return 0
            lax.fori_loop(0, NG, group, 0)
            hs = [pltpu.async_copy(hst0, h0_hbm.at[pl.ds(base * ROWW, CE)], sem),
                  pltpu.async_copy(hst1, h1_hbm.at[pl.ds(base * ROWW, CE)], sem)]
            for h in hs:
                h.wait()
            return 0
        lax.fori_loop(0, N_CHUNK, chunk, 0)

    @pl.when(c == 0)
    def _():
        run(la_hbm, ra_hbm, wea_hbm, affa_hbm, h00_hbm, h01_hbm)

    @pl.when(c == 1)
    def _():
        run(lb_hbm, rb_hbm, web_hbm, affb_hbm, h10_hbm, h11_hbm)


def _sc_hcompute(idxl, idxr, ef1, la, lb, ra, rb, wea, web, ea, affa, affb):
    kfn = functools.partial(
        pl.kernel, mesh=_mesh,
        compiler_params=pltpu.CompilerParams(use_tc_tiling_on_sc=False),
        out_type=tuple(jax.ShapeDtypeStruct((N_EDGE, 16), f32)
                       for _ in range(4)),
        scratch_types=[pltpu.VMEM((CHUNK_ROWS, ROWW), i32),
                       pltpu.VMEM((CHUNK_ROWS, ROWW), i32),
                       pltpu.VMEM((CE,), f32),
                       pltpu.VMEM((CE, 32), f32),
                       pltpu.VMEM((CE, 32), f32),
                       pltpu.VMEM((32,), f32),
                       pltpu.VMEM((32,), f32),
                       pltpu.VMEM((64,), f32),
                       pltpu.VMEM((CE, 16), f32),
                       pltpu.VMEM((CE, 16), f32),
                       pltpu.SemaphoreType.DMA],
    )
    return kfn(_sc_hcompute_fn)(idxl, idxr, ef1, la, lb, ra, rb,
                                wea, web, ea, affa, affb)


def _sc_scatter_fn(idq0_hbm, idq1_hbm,
                   h00_hbm, h01_hbm, h10_hbm, h11_hbm, z_hbm, ones_hbm,
                   s00_hbm, s01_hbm, s10_hbm, s11_hbm, cnt_hbm,
                   idxd_v, hst, ones_v, acc_sh, sem):
    c = lax.axis_index("c")
    s = lax.axis_index("s")
    idqs = (idq0_hbm, idq1_hbm)
    pltpu.sync_copy(ones_hbm, ones_v)

    def phase(src_hbm, idxd_hbm, dst_hbm, dst_base):
        zoff = s * (HPAD // N_TILE)
        pltpu.sync_copy(z_hbm.at[pl.ds(zoff, HPAD // N_TILE)],
                        acc_sh.at[pl.ds(zoff, HPAD // N_TILE)])
        plsc.subcore_barrier()

        def chunk(i, _):
            base = s * ROWS_PER_TILE + i * CHUNK_ROWS
            pre = [pltpu.async_copy(idxd_hbm.at[pl.ds(base, CHUNK_ROWS)],
                                    idxd_v, sem)]
            if src_hbm is not None:
                pre.append(pltpu.async_copy(
                    src_hbm.at[pl.ds(base * ROWW, CE)], hst, sem))
            for h in pre:
                h.wait()
            if src_hbm is not None:
                sc = [pltpu.async_copy(hst.at[pl.ds(j * ROWW, ROWW)],
                                       acc_sh.at[idxd_v.at[j]], sem, add=True)
                      for j in range(CHUNK_ROWS)]
            else:
                sc = [pltpu.async_copy(ones_v, acc_sh.at[idxd_v.at[j]],
                                       sem, add=True)
                      for j in range(CHUNK_ROWS)]
            for h in sc:
                h.wait()
            return 0
        lax.fori_loop(0, N_CHUNK, chunk, 0)
        plsc.subcore_barrier()

        @pl.when(s < 8)
        def _():
            coff = s * (HALF // 8)
            pltpu.sync_copy(
                acc_sh.at[pl.ds(coff, HALF // 8)],
                dst_hbm.at[pl.ds(dst_base + coff, HALF // 8)])
        plsc.subcore_barrier()

    def run(ha_hbm, hb_hbm, outa_hbm, outb_hbm, idc_hbm, cbase):
        for half in range(2):
            phase(ha_hbm, idqs[half], outa_hbm, half * HALF)
            phase(hb_hbm, idqs[half], outb_hbm, half * HALF)
        phase(None, idc_hbm, cnt_hbm, cbase)

    @pl.when(c == 0)
    def _():
        run(h00_hbm, h01_hbm, s00_hbm, s01_hbm, idq0_hbm, 0)

    @pl.when(c == 1)
    def _():
        run(h10_hbm, h11_hbm, s10_hbm, s11_hbm, idq1_hbm, HALF)


def _sc_scatter(idq, hs, zh, ones16):
    kfn = functools.partial(
        pl.kernel, mesh=_mesh,
        compiler_params=pltpu.CompilerParams(use_tc_tiling_on_sc=False),
        out_type=tuple(jax.ShapeDtypeStruct((N_NODE, 16), f32)
                       for _ in range(5)),
        scratch_types=[pltpu.VMEM((CHUNK_ROWS, ROWW), i32),
                       pltpu.VMEM((CE, 16), f32),
                       pltpu.VMEM((ROWW, 16), f32),
                       pltpu.VMEM_SHARED((HPAD, 16), f32),
                       pltpu.SemaphoreType.DMA],
    )
    return kfn(_sc_scatter_fn)(idq[0], idq[1], hs[0], hs[1], hs[2], hs[3],
                               zh, ones16)


def _sc_cand_fn(log_hbm, cand_hbm, out_hbm, cidx_v, rows_v, sem):
    c = lax.axis_index("c")
    s = lax.axis_index("s")
    w = s * 2 + c
    pltpu.sync_copy(cand_hbm.at[pl.ds(w * 128, 128)], cidx_v)
    pltpu.async_copy(log_hbm.at[cidx_v], rows_v, sem).wait()
    pltpu.sync_copy(rows_v, out_hbm.at[pl.ds(w * 128, 128)])


def _sc_cand(logits16, cand):
    kfn = functools.partial(
        pl.kernel, mesh=_mesh,
        compiler_params=pltpu.CompilerParams(use_tc_tiling_on_sc=False),
        out_type=jax.ShapeDtypeStruct((4096, 16), f32),
        scratch_types=[pltpu.VMEM((128,), i32),
                       pltpu.VMEM((128, 16), f32),
                       pltpu.SemaphoreType.DMA],
    )
    return kfn(_sc_cand_fn)(logits16, cand)


def kernel(constraint_features, edge_indices, edge_features, variable_features,
           candidates, constraints_per_sample, variables_per_sample,
           candidates_per_sample, params):
    p = params
    idx0 = edge_indices[0].reshape(EROWS, ROWW)
    idx1 = edge_indices[1].reshape(EROWS, ROWW)
    ef1 = edge_features.reshape(N_EDGE)

    cs, vs, ea = _tc_instats(constraint_features, variable_features,
                             edge_features.reshape(N_EDGE // 128, 128),
                             p['edge_bn_g'], p['edge_bn_b'])
    ea16 = ea[0]
    eb16 = ea[1]
    ea2 = jnp.concatenate([ea16, eb16])

    cf0, l1a, l1b = _tc_embed(
        constraint_features, cs, p['cons_bn_g'].reshape(1, 5),
        p['cons_bn_b'].reshape(1, 5), p['cons_W1'].T,
        p['cons_b1'].reshape(1, EMB), p['cons_W2'].T,
        p['cons_b2'].reshape(1, EMB),
        [(p['vc_Wl'].T, p['vc_bl'].reshape(1, EMB))])
    zerob = jnp.zeros((1, EMB), f32)
    vf0, r1a, r1b, r2a, r2b = _tc_embed(
        variable_features, vs, p['var_bn_g'].reshape(1, 19),
        p['var_bn_b'].reshape(1, 19), p['var_W1'].T,
        p['var_b1'].reshape(1, EMB), p['var_W2'].T,
        p['var_b2'].reshape(1, EMB),
        [(p['vc_Wr'].T, zerob), (p['cv_Wr'].T, zerob)])

    ones16 = jnp.ones((ROWW, 16), f32)
    zh = jnp.zeros((HPAD, 16), f32)

    def conv_pass(la, lb, ra, rb, we, fg, fb, idxd, finW, finb):
        wea, web = we[:32, 0], we[32:, 0]
        part = _sc_stats(idx0, idx1, ef1, la, lb, ra, rb, wea, web, ea2)
        aff = _tc_bnfin(part, fg, fb)
        affa = jnp.concatenate([aff[0, :32], aff[1, :32]])
        affb = jnp.concatenate([aff[0, 32:], aff[1, 32:]])
        idq = _tc_idxsplit(idxd)
        hs = _sc_hcompute(idx0, idx1, ef1, la, lb, ra, rb,
                          wea, web, ea2, affa, affb)
        out5 = _sc_scatter(idq, hs, zh, ones16)
        return _tc_convpre(out5[:4], out5[4], finW.T, finb)

    conv1, ps1 = conv_pass(l1a, l1b, r1a, r1b, p['vc_We'],
                           p['vc_fin_bn_g'], p['vc_fin_bn_b'],
                           idx0, p['vc_fin_W'], p['vc_fin_b'])
    row = lambda i: (i, 0)
    l2a, l2b = _tc_convpost(
        _post1_body, conv1, ps1, cf0, p['vc_post_bn_g'], p['vc_post_bn_b'],
        p['vc_o1_W'].T, p['vc_o1_b'], p['vc_o2_W'].T, p['vc_o2_b'],
        [p['cv_Wl'].T, p['cv_bl'].reshape(1, EMB)],
        (jax.ShapeDtypeStruct((N_NODE, 32), f32),
         jax.ShapeDtypeStruct((N_NODE, 32), f32)),
        (pl.BlockSpec((BLK, 32), row), pl.BlockSpec((BLK, 32), row)))

    conv2, ps2 = conv_pass(l2a, l2b, r2a, r2b, p['cv_We'],
                           p['cv_fin_bn_g'], p['cv_fin_bn_b'],
                           idx1, p['cv_fin_W'], p['cv_fin_b'])
    logits16 = _tc_convpost(
        _post2_body, conv2, ps2, vf0, p['cv_post_bn_g'], p['cv_post_bn_b'],
        p['cv_o1_W'].T, p['cv_o1_b'], p['cv_o2_W'].T, p['cv_o2_b'],
        [p['out_W1'].T, p['out_b1'].reshape(1, EMB),
         p['out_W2'].reshape(1, EMB), p['out_b2'].reshape(1, 1)],
        jax.ShapeDtypeStruct((N_NODE, 16), f32),
        pl.BlockSpec((BLK, 16), row))

    out = _sc_cand(logits16, candidates)
    return out[:, 0:1]
